# Initial kernel scaffold; baseline (speedup 1.0000x reference)
import jax
import jax.numpy as jnp
from jax.experimental import pallas as pl


def kernel(state_tensor, operation, prediction, set_values, set_masks):
    raise NotImplementedError("write your pallas kernel here")


# speedup vs baseline: 1.8805x; 1.8805x over previous
import functools

import jax
import jax.numpy as jnp
from jax import lax
from jax.experimental import pallas as pl
from jax.experimental.pallas import tpu as pltpu
from jax.experimental.pallas import tpu_sc as plsc

B = 262144
W = 64
WP = 2 * W
N_OPS = 1024
C = 128
LANES = 16


@functools.lru_cache(maxsize=None)
def _build_sc_kernel():
    info = plsc.get_sparse_core_info()
    num_cores, num_subcores = info.num_cores, info.num_subcores
    n_workers = num_cores * num_subcores
    rows_per_worker = B // n_workers
    n_chunks = rows_per_worker // C
    mesh = plsc.VectorSubcoreMesh(core_axis_name="c", subcore_axis_name="s")

    @functools.partial(
        pl.kernel,
        mesh=mesh,
        out_type=jax.ShapeDtypeStruct((B, WP), jnp.int32),
        scratch_types=[
            pltpu.VMEM((C,), jnp.int32),
            pltpu.VMEM((C, WP), jnp.int32),
            pltpu.VMEM((C, WP), jnp.int32),
            pltpu.SemaphoreType.DMA,
        ],
    )
    def sc_kernel(state_hbm, op_hbm, packed_hbm, out_hbm,
                  idx_v, srows_v, grows_v, sem):
        wid = lax.axis_index("s") * jnp.int32(num_cores) + lax.axis_index("c")
        wbase = wid * jnp.int32(rows_per_worker)

        def chunk_body(i, carry):
            base = wbase + i * jnp.int32(C)
            pltpu.sync_copy(op_hbm.at[pl.ds(base, C)], idx_v)
            gather = pltpu.async_copy(packed_hbm.at[idx_v], grows_v, sem)
            pltpu.sync_copy(state_hbm.at[pl.ds(base, C), :], srows_v)
            gather.wait()

            def row_body(r, rcarry):
                for l in range(WP // LANES):
                    s = srows_v[r, pl.ds(l * LANES, LANES)]
                    p = grows_v[r, pl.ds(l * LANES, LANES)]
                    o = jnp.where(p < 0, p & jnp.int32(0x7FFFFFFF), s)
                    srows_v[r, pl.ds(l * LANES, LANES)] = o
                return rcarry

            lax.fori_loop(jnp.int32(0), jnp.int32(C), row_body, jnp.int32(0))
            pltpu.sync_copy(srows_v, out_hbm.at[pl.ds(base, C), :])
            return carry

        lax.fori_loop(jnp.int32(0), jnp.int32(n_chunks), chunk_body,
                      jnp.int32(0))

    return sc_kernel


def kernel(state_tensor, operation, prediction, set_values, set_masks):
    del prediction
    state32 = lax.bitcast_convert_type(state_tensor, jnp.int32).reshape(B, WP)
    op32 = operation.astype(jnp.int32)
    sv32 = lax.bitcast_convert_type(set_values, jnp.int32)
    packed = jnp.where(set_masks[:, :, None],
                       sv32 | jnp.int32(-(2 ** 31)),
                       sv32).reshape(N_OPS, WP)
    out32 = _build_sc_kernel()(state32, op32, packed)
    return lax.bitcast_convert_type(out32.reshape(B, W, 2), jnp.int64)
